# baseline (speedup 1.0000x reference)
import functools

import jax
import jax.numpy as jnp
from jax.experimental import pallas as pl
from jax.experimental.pallas import tpu as pltpu

ALPHA = 0.2
LOG2E = 1.4426950408889634


def _layer_kernel(x_ref, w_ref, a1_ref, a2_ref, o_ref, wh_ref, g_ref,
                  *, bi, final):
    i = pl.program_id(0)

    @pl.when(i == 0)
    def _():
        wh = jnp.dot(x_ref[...], w_ref[...],
                     preferred_element_type=jnp.float32)
        wh_ref[...] = wh
        f2 = jax.lax.dot_general(a2_ref[...], wh, (((0,), (1,)), ((), ())),
                                 preferred_element_type=jnp.float32)
        g_ref[0:1, :] = f2 * LOG2E
        g_ref[1:2, :] = (ALPHA * LOG2E) * f2

    wh = wh_ref[...]
    wh_blk = wh_ref[pl.ds(i * bi, bi), :]
    f1 = jnp.dot(wh_blk, a1_ref[...],
                 preferred_element_type=jnp.float32)
    g1 = g_ref[0:1, :]
    g2 = g_ref[1:2, :]
    f2max = jnp.max(g1) * (1.0 / LOG2E)
    s = f1 + f2max
    m = jnp.maximum(s, ALPHA * s)
    c1 = (f1 - m) * LOG2E
    c2 = (ALPHA * f1 - m) * LOG2E
    w = jnp.exp2(jnp.maximum(c1 + g1, c2 + g2))
    acc = jnp.dot(w.astype(jnp.bfloat16), wh.astype(jnp.bfloat16),
                  preferred_element_type=jnp.float32)
    z = jnp.sum(w, axis=1, keepdims=True)
    h = acc / z
    h = jnp.where(h > 0, h, jnp.exp(jnp.minimum(h, 0.0)) - 1.0)
    if final:
        hm = jnp.max(h, axis=1, keepdims=True)
        lse = jnp.log(jnp.sum(jnp.exp(h - hm), axis=1, keepdims=True)) + hm
        h = h - lse
    o_ref[...] = h


def _gat_layer(x, w, a, *, final, bi=512):
    n = x.shape[0]
    fin, f = w.shape
    kfn = functools.partial(_layer_kernel, bi=bi, final=final)
    return pl.pallas_call(
        kfn,
        grid=(n // bi,),
        in_specs=[
            pl.BlockSpec((n, fin), lambda i: (0, 0)),
            pl.BlockSpec((fin, f), lambda i: (0, 0)),
            pl.BlockSpec((f, 1), lambda i: (0, 0)),
            pl.BlockSpec((f, 1), lambda i: (0, 0)),
        ],
        out_specs=pl.BlockSpec((bi, f), lambda i: (i, 0)),
        out_shape=jax.ShapeDtypeStruct((n, f), jnp.float32),
        scratch_shapes=[
            pltpu.VMEM((n, f), jnp.float32),
            pltpu.VMEM((2, n), jnp.float32),
        ],
    )(x, w, a[:f], a[f:])


def kernel(x, W0, a0, W_out, a_out):
    h1 = _gat_layer(x, W0, a0, final=False)
    return _gat_layer(h1, W_out, a_out, final=True)


# speedup vs baseline: 1.1360x; 1.1360x over previous
import functools

import jax
import jax.numpy as jnp
from jax.experimental import pallas as pl
from jax.experimental.pallas import tpu as pltpu

ALPHA = 0.2
LOG2E = 1.4426950408889634


def _layer_kernel(x_ref, w_ref, a1_ref, a2_ref, o_ref, wh_ref, g_ref,
                  *, bi, final):
    i = pl.program_id(0)

    @pl.when(i == 0)
    def _():
        wh = jnp.dot(x_ref[...], w_ref[...],
                     preferred_element_type=jnp.float32)
        wh_ref[...] = wh
        f2 = jax.lax.dot_general(a2_ref[...], wh, (((0,), (1,)), ((), ())),
                                 preferred_element_type=jnp.float32)
        g_ref[0:1, :] = f2 * LOG2E
        g_ref[1:2, :] = (ALPHA * LOG2E) * f2

    n = wh_ref.shape[0]
    f = wh_ref.shape[1]
    wh_blk = wh_ref[pl.ds(i * bi, bi), :]
    f1 = jnp.dot(wh_blk, a1_ref[...],
                 preferred_element_type=jnp.float32)
    f2max = jnp.max(g_ref[0:1, :]) * (1.0 / LOG2E)
    s = f1 + f2max
    m = jnp.maximum(s, ALPHA * s)
    c1 = (f1 - m) * LOG2E
    c2 = (ALPHA * f1 - m) * LOG2E
    cj = 512
    acc = jnp.zeros((bi, f), jnp.float32)
    z = jnp.zeros((bi, 1), jnp.float32)
    for jc in range(n // cj):
        g1 = g_ref[0:1, pl.ds(jc * cj, cj)]
        g2 = g_ref[1:2, pl.ds(jc * cj, cj)]
        w = jnp.exp2(jnp.maximum(c1 + g1, c2 + g2))
        acc = acc + jnp.dot(w, wh_ref[pl.ds(jc * cj, cj), :],
                            preferred_element_type=jnp.float32)
        z = z + jnp.sum(w, axis=1, keepdims=True)
    h = acc / z
    h = jnp.where(h > 0, h, jnp.exp(jnp.minimum(h, 0.0)) - 1.0)
    if final:
        hm = jnp.max(h, axis=1, keepdims=True)
        lse = jnp.log(jnp.sum(jnp.exp(h - hm), axis=1, keepdims=True)) + hm
        h = h - lse
    o_ref[...] = h


def _gat_layer(x, w, a, *, final, bi=512):
    n = x.shape[0]
    fin, f = w.shape
    kfn = functools.partial(_layer_kernel, bi=bi, final=final)
    return pl.pallas_call(
        kfn,
        grid=(n // bi,),
        in_specs=[
            pl.BlockSpec((n, fin), lambda i: (0, 0)),
            pl.BlockSpec((fin, f), lambda i: (0, 0)),
            pl.BlockSpec((f, 1), lambda i: (0, 0)),
            pl.BlockSpec((f, 1), lambda i: (0, 0)),
        ],
        out_specs=pl.BlockSpec((bi, f), lambda i: (i, 0)),
        out_shape=jax.ShapeDtypeStruct((n, f), jnp.float32),
        scratch_shapes=[
            pltpu.VMEM((n, f), jnp.float32),
            pltpu.VMEM((2, n), jnp.float32),
        ],
    )(x, w, a[:f], a[f:])


def kernel(x, W0, a0, W_out, a_out):
    h1 = _gat_layer(x, W0, a0, final=False)
    return _gat_layer(h1, W_out, a_out, final=True)
